# baseline (speedup 1.0000x reference)
import dataclasses

import jax
import jax.numpy as jnp
from jax import lax
from jax.experimental import pallas as pl
from jax.experimental.pallas import tpu as pltpu
from jax.experimental.pallas import tpu_sc as plsc

N = 10000
E = 320000
D = 128

NC = 2
NS = 16
NW = NC * NS
B = 128
BD = 128
NP = 10240
EP = NW * NP
EPT = EP // NW
NCH = EPT // B
ROWS = NP // NS
RR = NP // 128

_mesh = plsc.VectorSubcoreMesh(core_axis_name="c", subcore_axis_name="s")

_cp = pltpu.CompilerParams()
if "needs_layout_passes" in pltpu.CompilerParams.__dataclass_fields__:
    _cp = dataclasses.replace(_cp, needs_layout_passes=False)


def _sc_degree_body(dst_hbm, deg_hbm, idx_v, hist_v):
    cid = lax.axis_index("c")
    sid = lax.axis_index("s")
    wid = sid * NC + cid

    @pl.loop(0, RR)
    def _(i):
        @pl.loop(0, 128, step=16)
        def _(j):
            hist_v[i, pl.ds(j, 16)] = jnp.zeros((16,), jnp.float32)

    pltpu.sync_copy(dst_hbm.at[wid], idx_v)
    ones16 = jnp.full((16,), 1.0, jnp.float32)

    @pl.loop(0, EP // NW // BD)
    def _(j):
        @pl.loop(0, BD, step=16)
        def _(k):
            idx16 = idx_v[j, pl.ds(k, 16)]
            hi = lax.shift_right_logical(idx16, 7)
            lo = lax.bitwise_and(idx16, 127)
            plsc.addupdate_scatter(hist_v, [hi, lo], ones16)

    pltpu.sync_copy(hist_v, deg_hbm.at[wid])


@jax.jit
def _sc_degree(dst_idx):
    kern = pl.kernel(
        _sc_degree_body,
        out_type=jax.ShapeDtypeStruct((NW, RR, 128), jnp.float32),
        mesh=_mesh,
        compiler_params=_cp,
        scratch_types=[
            pltpu.VMEM((EP // NW // BD, BD), jnp.int32),
            pltpu.VMEM((RR, 128), jnp.float32),
        ],
    )
    return kern(dst_idx)


def _unpack_chunk(pidx_v, jc, su, du):
    @pl.loop(0, B, step=16)
    def _(k):
        v = pidx_v[jc, pl.ds(k, 16)]
        su[pl.ds(k, 16)] = lax.bitwise_and(v, 0xFFFF)
        du[pl.ds(k, 16)] = lax.shift_right_logical(v, 16)


def _sc_scatter_body(y_hbm, pidx_hbm, out_hbm,
                     pidx_v, src0, dst0, src1, dst1, buf0, buf1, acc_sh,
                     gsem0, gsem1, ssem0, ssem1):
    cid = lax.axis_index("c")
    sid = lax.axis_index("s")
    wid = sid * NC + cid

    @pl.loop(0, B)
    def _(i):
        @pl.loop(0, D, step=16)
        def _(j):
            buf0[i, pl.ds(j, 16)] = jnp.zeros((16,), jnp.float32)

    @pl.loop(0, ROWS, step=B)
    def _(r):
        pltpu.sync_copy(buf0, acc_sh.at[pl.ds(sid * ROWS + r, B)])

    pltpu.sync_copy(pidx_hbm.at[wid], pidx_v)
    _unpack_chunk(pidx_v, 0, src0, dst0)
    _unpack_chunk(pidx_v, 1, src1, dst1)

    pltpu.async_copy(y_hbm.at[src0], buf0, gsem0)
    pltpu.async_copy(y_hbm.at[src1], buf1, gsem1)
    plsc.subcore_barrier()

    @pl.loop(0, NCH, step=2)
    def _(j):
        pltpu.make_async_copy(y_hbm.at[src0], buf0, gsem0).wait()
        pltpu.async_copy(buf0, acc_sh.at[dst0], ssem0, add=True)

        pltpu.make_async_copy(y_hbm.at[src1], buf1, gsem1).wait()
        pltpu.async_copy(buf1, acc_sh.at[dst1], ssem1, add=True)

        @pl.when(j + 2 < NCH)
        def _():
            pltpu.make_async_copy(buf0, acc_sh.at[dst0], ssem0).wait()
            _unpack_chunk(pidx_v, j + 2, src0, dst0)
            pltpu.async_copy(y_hbm.at[src0], buf0, gsem0)

        @pl.when(j + 3 < NCH)
        def _():
            pltpu.make_async_copy(buf1, acc_sh.at[dst1], ssem1).wait()
            _unpack_chunk(pidx_v, j + 3, src1, dst1)
            pltpu.async_copy(y_hbm.at[src1], buf1, gsem1)

    pltpu.make_async_copy(buf0, acc_sh.at[dst0], ssem0).wait()
    pltpu.make_async_copy(buf1, acc_sh.at[dst1], ssem1).wait()
    plsc.subcore_barrier()
    pltpu.sync_copy(acc_sh.at[pl.ds(sid * ROWS, ROWS)],
                    out_hbm.at[cid, pl.ds(sid * ROWS, ROWS)])


@jax.jit
def _sc_scatter(y, pidx):
    kern = pl.kernel(
        _sc_scatter_body,
        out_type=jax.ShapeDtypeStruct((NC, NP, D), jnp.float32),
        mesh=_mesh,
        compiler_params=_cp,
        scratch_types=[
            pltpu.VMEM((NCH, B), jnp.int32),
            pltpu.VMEM((B,), jnp.int32),
            pltpu.VMEM((B,), jnp.int32),
            pltpu.VMEM((B,), jnp.int32),
            pltpu.VMEM((B,), jnp.int32),
            pltpu.VMEM((B, D), jnp.float32),
            pltpu.VMEM((B, D), jnp.float32),
            pltpu.VMEM_SHARED((NP, D), jnp.float32),
            pltpu.SemaphoreType.DMA,
            pltpu.SemaphoreType.DMA,
            pltpu.SemaphoreType.DMA,
            pltpu.SemaphoreType.DMA,
        ],
    )
    return kern(y, pidx)


def _deginv_body(degp_ref, dinv2d_ref):
    deg = jnp.sum(degp_ref[...], axis=0) + 1.0
    dinv2d_ref[...] = lax.rsqrt(deg)


def _tc_deginv(degp):
    return pl.pallas_call(
        _deginv_body,
        out_shape=jax.ShapeDtypeStruct((RR, 128), jnp.float32),
    )(degp)


def _scale_body(dinv_ref, x_ref, w_ref, y_ref):
    xw = jnp.dot(x_ref[...], w_ref[...], preferred_element_type=jnp.float32)
    y_ref[...] = xw * dinv_ref[...]


def _tc_scale(dinv, x, w):
    return pl.pallas_call(
        _scale_body,
        out_shape=jax.ShapeDtypeStruct((N, D), jnp.float32),
    )(dinv, x, w)


def _mid_body(acc_ref, y_ref, dinv_ref, b_ref, w_ref, o_ref):
    dinv = dinv_ref[...]
    h = dinv * (acc_ref[0][:N] + acc_ref[1][:N] + y_ref[...]) + b_ref[...]
    h = jnp.where(h >= 0, h, 0.01 * h)
    o_ref[...] = jnp.dot(h, w_ref[...],
                         preferred_element_type=jnp.float32) * dinv


def _tc_mid(accp, y, dinv, b, w):
    return pl.pallas_call(
        _mid_body,
        out_shape=jax.ShapeDtypeStruct((N, D), jnp.float32),
    )(accp, y, dinv, b, w)


def _final_body(acc_ref, y_ref, dinv_ref, b3_ref,
                wf1_ref, bf1_ref, wf2_ref, bf2_ref, wf3_ref, bf3_ref, o_ref):
    h = (dinv_ref[...] * (acc_ref[0][:N] + acc_ref[1][:N] + y_ref[...])
         + b3_ref[...])
    pooled = jnp.concatenate(
        [jnp.mean(h[i * 400:(i + 1) * 400, :], axis=0, keepdims=True)
         for i in range(25)], axis=0)
    g = jnp.dot(pooled, wf1_ref[...],
                preferred_element_type=jnp.float32) + bf1_ref[...]
    g = jnp.where(g >= 0, g, 0.01 * g)
    g = jnp.dot(g, wf2_ref[...],
                preferred_element_type=jnp.float32) + bf2_ref[...]
    g = jnp.where(g >= 0, g, 0.01 * g)
    o_ref[...] = jnp.dot(g, wf3_ref[...],
                         preferred_element_type=jnp.float32) + bf3_ref[...]


def _tc_final(accp, y, dinv, b3, wf1, bf1, wf2, bf2, wf3, bf3):
    return pl.pallas_call(
        _final_body,
        out_shape=jax.ShapeDtypeStruct((25, 16), jnp.float32),
    )(accp, y, dinv, b3, wf1, bf1, wf2, bf2, wf3, bf3)


def kernel(x, edge_index, W1, b1, W2, b2, W3, b3,
           Wf1, bf1, Wf2, bf2, Wf3, bf3):
    ei = edge_index.astype(jnp.int32)
    npad = EP - E
    pad_src = jnp.arange(npad, dtype=jnp.int32) % N
    pad_dst = N + (jnp.arange(npad, dtype=jnp.int32) % (NP - N))
    src_full = jnp.concatenate([ei[0], pad_src])
    dst_full = jnp.concatenate([ei[1], pad_dst])
    pidx = (src_full | (dst_full << 16)).reshape(NW, NCH, B)
    dst_deg = dst_full.reshape(NW, EP // NW // BD, BD)
    b1r = b1.reshape(1, -1)
    b2r = b2.reshape(1, -1)
    b3r = b3.reshape(1, -1)
    bf1r = bf1.reshape(1, -1)
    bf2r = bf2.reshape(1, -1)
    bf3r = bf3.reshape(1, -1)

    degp = _sc_degree(dst_deg)
    dinv2d = _tc_deginv(degp)
    dinv = dinv2d.reshape(NP, 1)[:N]
    y1 = _tc_scale(dinv, x, W1)

    acc1 = _sc_scatter(y1, pidx)
    y2 = _tc_mid(acc1, y1, dinv, b1r, W2)
    acc2 = _sc_scatter(y2, pidx)
    y3 = _tc_mid(acc2, y2, dinv, b2r, W3)
    acc3 = _sc_scatter(y3, pidx)
    return _tc_final(acc3, y3, dinv, b3r, Wf1, bf1r, Wf2, bf2r, Wf3, bf3r)


# speedup vs baseline: 1.3275x; 1.3275x over previous
import dataclasses

import jax
import jax.numpy as jnp
from jax import lax
from jax.experimental import pallas as pl
from jax.experimental.pallas import tpu as pltpu
from jax.experimental.pallas import tpu_sc as plsc

N = 10000
E = 320000
D = 128

NC = 2
NS = 16
NW = NC * NS
B = 128
BD = 128
NP = 10240
EP = NW * NP
EPT = EP // NW
NCH = EPT // B
ROWS = NP // NS
RR = NP // 128

_mesh = plsc.VectorSubcoreMesh(core_axis_name="c", subcore_axis_name="s")

_cp = pltpu.CompilerParams()
if "needs_layout_passes" in pltpu.CompilerParams.__dataclass_fields__:
    _cp = dataclasses.replace(_cp, needs_layout_passes=False)


EFULL = EPT * (NW - 1)
ELAST = E - EFULL
NPADE = EP - E


def _prep_groups(ebuf, hist_v, pk_v, ngroups):
    ones16 = jnp.full((16,), 1.0, jnp.float32)

    @pl.loop(0, ngroups)
    def _(g):
        p = g * 16
        s16 = ebuf[0, pl.ds(p, 16)]
        d16 = ebuf[1, pl.ds(p, 16)]
        hi = lax.shift_right_logical(d16, 7)
        lo = lax.bitwise_and(d16, 127)
        plsc.addupdate_scatter(hist_v, [hi, lo], ones16)
        pk_v[pl.ds(p, 16)] = lax.bitwise_or(s16, lax.shift_left(d16, 16))


def _sc_prep_body(edge_hbm, deg_hbm, pidx_hbm, ebuf, hist_v, pk_v):
    cid = lax.axis_index("c")
    sid = lax.axis_index("s")
    wid = sid * NC + cid

    @pl.loop(0, RR)
    def _(i):
        @pl.loop(0, 128, step=16)
        def _(j):
            hist_v[i, pl.ds(j, 16)] = jnp.zeros((16,), jnp.float32)

    @pl.when(wid != NW - 1)
    def _():
        pltpu.sync_copy(edge_hbm.at[:, pl.ds(wid * EPT, EPT)], ebuf)
        _prep_groups(ebuf, hist_v, pk_v, EPT // 16)

    @pl.when(wid == NW - 1)
    def _():
        pltpu.sync_copy(edge_hbm.at[:, pl.ds(EFULL, ELAST)],
                        ebuf.at[:, pl.ds(0, ELAST)])
        _prep_groups(ebuf, hist_v, pk_v, ELAST // 16)

        @pl.loop(0, NPADE // 16)
        def _(g):
            ii = lax.iota(jnp.int32, 16) + g * 16
            s = lax.rem(ii * 13 + 1, N)
            dpad = N + lax.rem(ii, NP - N)
            pk_v[pl.ds(ELAST + g * 16, 16)] = lax.bitwise_or(
                s, lax.shift_left(dpad, 16))

    pltpu.sync_copy(hist_v, deg_hbm.at[wid])
    pltpu.sync_copy(pk_v, pidx_hbm.at[pl.ds(wid * EPT, EPT)])


@jax.jit
def _sc_prep(edge_index):
    kern = pl.kernel(
        _sc_prep_body,
        out_type=[
            jax.ShapeDtypeStruct((NW, RR, 128), jnp.float32),
            jax.ShapeDtypeStruct((EP,), jnp.int32),
        ],
        mesh=_mesh,
        compiler_params=_cp,
        scratch_types=[
            pltpu.VMEM((2, EPT), jnp.int32),
            pltpu.VMEM((RR, 128), jnp.float32),
            pltpu.VMEM((EPT,), jnp.int32),
        ],
    )
    return kern(edge_index)


def _unpack_chunk(pidx_v, jc, su, du):
    @pl.loop(0, B, step=16)
    def _(k):
        v = pidx_v[pl.ds(jc * B + k, 16)]
        su[pl.ds(k, 16)] = lax.bitwise_and(v, 0xFFFF)
        du[pl.ds(k, 16)] = lax.shift_right_logical(v, 16)


def _sc_scatter_body(y_hbm, pidx_hbm, out_hbm,
                     pidx_v, src0, dst0, src1, dst1, buf0, buf1, acc_sh,
                     gsem0, gsem1):
    cid = lax.axis_index("c")
    sid = lax.axis_index("s")
    wid = sid * NC + cid

    @pl.loop(0, B)
    def _(i):
        @pl.loop(0, D, step=16)
        def _(j):
            buf0[i, pl.ds(j, 16)] = jnp.zeros((16,), jnp.float32)

    @pl.loop(0, ROWS, step=B)
    def _(r):
        pltpu.sync_copy(buf0, acc_sh.at[pl.ds(sid * ROWS + r, B)])

    pltpu.sync_copy(pidx_hbm.at[pl.ds(wid * EPT, EPT)], pidx_v)
    _unpack_chunk(pidx_v, 0, src0, dst0)
    _unpack_chunk(pidx_v, 1, src1, dst1)

    pltpu.async_copy(y_hbm.at[src0], buf0, gsem0)
    pltpu.async_copy(y_hbm.at[src1], buf1, gsem1)
    plsc.subcore_barrier()

    @pl.loop(0, NCH, step=2)
    def _(j):
        pltpu.make_async_copy(y_hbm.at[src0], buf0, gsem0).wait()
        pltpu.sync_copy(buf0, acc_sh.at[dst0], add=True)

        @pl.when(j + 2 < NCH)
        def _():
            _unpack_chunk(pidx_v, j + 2, src0, dst0)
            pltpu.async_copy(y_hbm.at[src0], buf0, gsem0)

        pltpu.make_async_copy(y_hbm.at[src1], buf1, gsem1).wait()
        pltpu.sync_copy(buf1, acc_sh.at[dst1], add=True)

        @pl.when(j + 3 < NCH)
        def _():
            _unpack_chunk(pidx_v, j + 3, src1, dst1)
            pltpu.async_copy(y_hbm.at[src1], buf1, gsem1)

    plsc.subcore_barrier()
    pltpu.sync_copy(acc_sh.at[pl.ds(sid * ROWS, ROWS)],
                    out_hbm.at[cid, pl.ds(sid * ROWS, ROWS)])


@jax.jit
def _sc_scatter(y, pidx):
    kern = pl.kernel(
        _sc_scatter_body,
        out_type=jax.ShapeDtypeStruct((NC, NP, D), jnp.float32),
        mesh=_mesh,
        compiler_params=_cp,
        scratch_types=[
            pltpu.VMEM((EPT,), jnp.int32),
            pltpu.VMEM((B,), jnp.int32),
            pltpu.VMEM((B,), jnp.int32),
            pltpu.VMEM((B,), jnp.int32),
            pltpu.VMEM((B,), jnp.int32),
            pltpu.VMEM((B, D), jnp.float32),
            pltpu.VMEM((B, D), jnp.float32),
            pltpu.VMEM_SHARED((NP, D), jnp.float32),
            pltpu.SemaphoreType.DMA,
            pltpu.SemaphoreType.DMA,
        ],
    )
    return kern(y, pidx)


def _deginv_body(degp_ref, dinv2d_ref):
    deg = jnp.sum(degp_ref[...], axis=0) + 1.0
    dinv2d_ref[...] = lax.rsqrt(deg)


def _tc_deginv(degp):
    return pl.pallas_call(
        _deginv_body,
        out_shape=jax.ShapeDtypeStruct((RR, 128), jnp.float32),
    )(degp)


def _scale_body(dinv_ref, x_ref, w_ref, y_ref):
    xw = jnp.dot(x_ref[...], w_ref[...], preferred_element_type=jnp.float32)
    y_ref[...] = xw * dinv_ref[...]


def _tc_scale(dinv, x, w):
    return pl.pallas_call(
        _scale_body,
        out_shape=jax.ShapeDtypeStruct((N, D), jnp.float32),
    )(dinv, x, w)


def _mid_body(acc_ref, y_ref, dinv_ref, b_ref, w_ref, o_ref):
    dinv = dinv_ref[...]
    h = dinv * (acc_ref[0][:N] + acc_ref[1][:N] + y_ref[...]) + b_ref[...]
    h = jnp.where(h >= 0, h, 0.01 * h)
    o_ref[...] = jnp.dot(h, w_ref[...],
                         preferred_element_type=jnp.float32) * dinv


def _tc_mid(accp, y, dinv, b, w):
    return pl.pallas_call(
        _mid_body,
        out_shape=jax.ShapeDtypeStruct((N, D), jnp.float32),
    )(accp, y, dinv, b, w)


def _final_body(acc_ref, y_ref, dinv_ref, b3_ref,
                wf1_ref, bf1_ref, wf2_ref, bf2_ref, wf3_ref, bf3_ref, o_ref):
    h = (dinv_ref[...] * (acc_ref[0][:N] + acc_ref[1][:N] + y_ref[...])
         + b3_ref[...])
    pooled = jnp.concatenate(
        [jnp.mean(h[i * 400:(i + 1) * 400, :], axis=0, keepdims=True)
         for i in range(25)], axis=0)
    g = jnp.dot(pooled, wf1_ref[...],
                preferred_element_type=jnp.float32) + bf1_ref[...]
    g = jnp.where(g >= 0, g, 0.01 * g)
    g = jnp.dot(g, wf2_ref[...],
                preferred_element_type=jnp.float32) + bf2_ref[...]
    g = jnp.where(g >= 0, g, 0.01 * g)
    o_ref[...] = jnp.dot(g, wf3_ref[...],
                         preferred_element_type=jnp.float32) + bf3_ref[...]


def _tc_final(accp, y, dinv, b3, wf1, bf1, wf2, bf2, wf3, bf3):
    return pl.pallas_call(
        _final_body,
        out_shape=jax.ShapeDtypeStruct((25, 16), jnp.float32),
    )(accp, y, dinv, b3, wf1, bf1, wf2, bf2, wf3, bf3)


def kernel(x, edge_index, W1, b1, W2, b2, W3, b3,
           Wf1, bf1, Wf2, bf2, Wf3, bf3):
    ei = edge_index.astype(jnp.int32)
    b1r = b1.reshape(1, -1)
    b2r = b2.reshape(1, -1)
    b3r = b3.reshape(1, -1)
    bf1r = bf1.reshape(1, -1)
    bf2r = bf2.reshape(1, -1)
    bf3r = bf3.reshape(1, -1)

    degp, pidx = _sc_prep(ei)
    dinv2d = _tc_deginv(degp)
    dinv = dinv2d.reshape(NP, 1)[:N]
    y1 = _tc_scale(dinv, x, W1)

    acc1 = _sc_scatter(y1, pidx)
    y2 = _tc_mid(acc1, y1, dinv, b1r, W2)
    acc2 = _sc_scatter(y2, pidx)
    y3 = _tc_mid(acc2, y2, dinv, b2r, W3)
    acc3 = _sc_scatter(y3, pidx)
    return _tc_final(acc3, y3, dinv, b3r, Wf1, bf1r, Wf2, bf2r, Wf3, bf3r)


# speedup vs baseline: 1.4905x; 1.1228x over previous
import dataclasses

import jax
import jax.numpy as jnp
from jax import lax
from jax.experimental import pallas as pl
from jax.experimental.pallas import tpu as pltpu
from jax.experimental.pallas import tpu_sc as plsc

N = 10000
E = 320000
D = 128

NC = 2
NS = 16
NW = NC * NS
B = 64
BD = 128
NP = 10240
EP = NW * NP
EPT = EP // NW
NCH = EPT // B
ROWS = NP // NS
RR = NP // 128

_mesh = plsc.VectorSubcoreMesh(core_axis_name="c", subcore_axis_name="s")

_cp = pltpu.CompilerParams()
if "needs_layout_passes" in pltpu.CompilerParams.__dataclass_fields__:
    _cp = dataclasses.replace(_cp, needs_layout_passes=False)


EFULL = EPT * (NW - 1)
ELAST = E - EFULL
NPADE = EP - E


def _prep_groups(ebuf, hist_v, pk_v, ngroups):
    ones16 = jnp.full((16,), 1.0, jnp.float32)

    @pl.loop(0, ngroups)
    def _(g):
        p = g * 16
        s16 = ebuf[0, pl.ds(p, 16)]
        d16 = ebuf[1, pl.ds(p, 16)]
        hi = lax.shift_right_logical(d16, 7)
        lo = lax.bitwise_and(d16, 127)
        plsc.addupdate_scatter(hist_v, [hi, lo], ones16)
        pk_v[pl.ds(p, 16)] = lax.bitwise_or(s16, lax.shift_left(d16, 16))


def _sc_prep_body(edge_hbm, deg_hbm, pidx_hbm, ebuf, hist_v, pk_v):
    cid = lax.axis_index("c")
    sid = lax.axis_index("s")
    wid = sid * NC + cid

    @pl.loop(0, RR)
    def _(i):
        @pl.loop(0, 128, step=16)
        def _(j):
            hist_v[i, pl.ds(j, 16)] = jnp.zeros((16,), jnp.float32)

    @pl.when(wid != NW - 1)
    def _():
        pltpu.sync_copy(edge_hbm.at[:, pl.ds(wid * EPT, EPT)], ebuf)
        _prep_groups(ebuf, hist_v, pk_v, EPT // 16)

    @pl.when(wid == NW - 1)
    def _():
        pltpu.sync_copy(edge_hbm.at[:, pl.ds(EFULL, ELAST)],
                        ebuf.at[:, pl.ds(0, ELAST)])
        _prep_groups(ebuf, hist_v, pk_v, ELAST // 16)

        @pl.loop(0, NPADE // 16)
        def _(g):
            ii = lax.iota(jnp.int32, 16) + g * 16
            s = lax.rem(ii * 13 + 1, N)
            dpad = N + lax.rem(ii, NP - N)
            pk_v[pl.ds(ELAST + g * 16, 16)] = lax.bitwise_or(
                s, lax.shift_left(dpad, 16))

    pltpu.sync_copy(hist_v, deg_hbm.at[wid])
    pltpu.sync_copy(pk_v, pidx_hbm.at[pl.ds(wid * EPT, EPT)])


@jax.jit
def _sc_prep(edge_index):
    kern = pl.kernel(
        _sc_prep_body,
        out_type=[
            jax.ShapeDtypeStruct((NW, RR, 128), jnp.float32),
            jax.ShapeDtypeStruct((EP,), jnp.int32),
        ],
        mesh=_mesh,
        compiler_params=_cp,
        scratch_types=[
            pltpu.VMEM((2, EPT), jnp.int32),
            pltpu.VMEM((RR, 128), jnp.float32),
            pltpu.VMEM((EPT,), jnp.int32),
        ],
    )
    return kern(edge_index)


def _unpack_chunk(pidx_v, jc, su, du):
    @pl.loop(0, B, step=16)
    def _(k):
        v = pidx_v[pl.ds(jc * B + k, 16)]
        su[pl.ds(k, 16)] = lax.bitwise_and(v, 0xFFFF)
        du[pl.ds(k, 16)] = lax.shift_right_logical(v, 16)


NBUF = 4


def _sc_scatter_body(y_hbm, pidx_hbm, out_hbm, pidx_v, *rest):
    srcs = rest[0:NBUF]
    dsts = rest[NBUF:2 * NBUF]
    bufs = rest[2 * NBUF:3 * NBUF]
    acc_sh = rest[3 * NBUF]
    gsems = rest[3 * NBUF + 1:]
    cid = lax.axis_index("c")
    sid = lax.axis_index("s")
    wid = sid * NC + cid

    @pl.loop(0, B)
    def _(i):
        @pl.loop(0, D, step=16)
        def _(j):
            bufs[0][i, pl.ds(j, 16)] = jnp.zeros((16,), jnp.float32)

    @pl.loop(0, ROWS, step=B)
    def _(r):
        pltpu.sync_copy(bufs[0], acc_sh.at[pl.ds(sid * ROWS + r, B)])

    pltpu.sync_copy(pidx_hbm.at[pl.ds(wid * EPT, EPT)], pidx_v)
    for t in range(NBUF):
        _unpack_chunk(pidx_v, t, srcs[t], dsts[t])
        pltpu.async_copy(y_hbm.at[srcs[t]], bufs[t], gsems[t])
    plsc.subcore_barrier()

    @pl.loop(0, NCH, step=NBUF)
    def _(j):
        for t in range(NBUF):
            pltpu.make_async_copy(y_hbm.at[srcs[t]], bufs[t],
                                  gsems[t]).wait()
            pltpu.sync_copy(bufs[t], acc_sh.at[dsts[t]], add=True)

            @pl.when(j + t + NBUF < NCH)
            def _():
                _unpack_chunk(pidx_v, j + t + NBUF, srcs[t], dsts[t])
                pltpu.async_copy(y_hbm.at[srcs[t]], bufs[t], gsems[t])

    plsc.subcore_barrier()
    pltpu.sync_copy(acc_sh.at[pl.ds(sid * ROWS, ROWS)],
                    out_hbm.at[cid, pl.ds(sid * ROWS, ROWS)])


@jax.jit
def _sc_scatter(y, pidx):
    kern = pl.kernel(
        _sc_scatter_body,
        out_type=jax.ShapeDtypeStruct((NC, NP, D), jnp.float32),
        mesh=_mesh,
        compiler_params=_cp,
        scratch_types=(
            [pltpu.VMEM((EPT,), jnp.int32)]
            + [pltpu.VMEM((B,), jnp.int32) for _ in range(2 * NBUF)]
            + [pltpu.VMEM((B, D), jnp.float32) for _ in range(NBUF)]
            + [pltpu.VMEM_SHARED((NP, D), jnp.float32)]
            + [pltpu.SemaphoreType.DMA for _ in range(NBUF)]
        ),
    )
    return kern(y, pidx)


def _deginv_body(degp_ref, dinv2d_ref):
    deg = jnp.sum(degp_ref[...], axis=0) + 1.0
    dinv2d_ref[...] = lax.rsqrt(deg)


def _tc_deginv(degp):
    return pl.pallas_call(
        _deginv_body,
        out_shape=jax.ShapeDtypeStruct((RR, 128), jnp.float32),
    )(degp)


def _scale_body(dinv_ref, x_ref, w_ref, y_ref):
    xw = jnp.dot(x_ref[...], w_ref[...], preferred_element_type=jnp.float32)
    y_ref[...] = xw * dinv_ref[...]


def _tc_scale(dinv, x, w):
    return pl.pallas_call(
        _scale_body,
        out_shape=jax.ShapeDtypeStruct((N, D), jnp.float32),
    )(dinv, x, w)


def _mid_body(acc_ref, y_ref, dinv_ref, b_ref, w_ref, o_ref):
    dinv = dinv_ref[...]
    h = dinv * (acc_ref[0][:N] + acc_ref[1][:N] + y_ref[...]) + b_ref[...]
    h = jnp.where(h >= 0, h, 0.01 * h)
    o_ref[...] = jnp.dot(h, w_ref[...],
                         preferred_element_type=jnp.float32) * dinv


def _tc_mid(accp, y, dinv, b, w):
    return pl.pallas_call(
        _mid_body,
        out_shape=jax.ShapeDtypeStruct((N, D), jnp.float32),
    )(accp, y, dinv, b, w)


def _final_body(acc_ref, y_ref, dinv_ref, b3_ref,
                wf1_ref, bf1_ref, wf2_ref, bf2_ref, wf3_ref, bf3_ref, o_ref):
    h = (dinv_ref[...] * (acc_ref[0][:N] + acc_ref[1][:N] + y_ref[...])
         + b3_ref[...])
    pooled = jnp.concatenate(
        [jnp.mean(h[i * 400:(i + 1) * 400, :], axis=0, keepdims=True)
         for i in range(25)], axis=0)
    g = jnp.dot(pooled, wf1_ref[...],
                preferred_element_type=jnp.float32) + bf1_ref[...]
    g = jnp.where(g >= 0, g, 0.01 * g)
    g = jnp.dot(g, wf2_ref[...],
                preferred_element_type=jnp.float32) + bf2_ref[...]
    g = jnp.where(g >= 0, g, 0.01 * g)
    o_ref[...] = jnp.dot(g, wf3_ref[...],
                         preferred_element_type=jnp.float32) + bf3_ref[...]


def _tc_final(accp, y, dinv, b3, wf1, bf1, wf2, bf2, wf3, bf3):
    return pl.pallas_call(
        _final_body,
        out_shape=jax.ShapeDtypeStruct((25, 16), jnp.float32),
    )(accp, y, dinv, b3, wf1, bf1, wf2, bf2, wf3, bf3)


def kernel(x, edge_index, W1, b1, W2, b2, W3, b3,
           Wf1, bf1, Wf2, bf2, Wf3, bf3):
    ei = edge_index.astype(jnp.int32)
    b1r = b1.reshape(1, -1)
    b2r = b2.reshape(1, -1)
    b3r = b3.reshape(1, -1)
    bf1r = bf1.reshape(1, -1)
    bf2r = bf2.reshape(1, -1)
    bf3r = bf3.reshape(1, -1)

    degp, pidx = _sc_prep(ei)
    dinv2d = _tc_deginv(degp)
    dinv = dinv2d.reshape(NP, 1)[:N]
    y1 = _tc_scale(dinv, x, W1)

    acc1 = _sc_scatter(y1, pidx)
    y2 = _tc_mid(acc1, y1, dinv, b1r, W2)
    acc2 = _sc_scatter(y2, pidx)
    y3 = _tc_mid(acc2, y2, dinv, b2r, W3)
    acc3 = _sc_scatter(y3, pidx)
    return _tc_final(acc3, y3, dinv, b3r, Wf1, bf1r, Wf2, bf2r, Wf3, bf3r)
